# baseline (speedup 1.0000x reference)
import jax
import jax.numpy as jnp
from jax.experimental import pallas as pl
from jax.experimental.pallas import tpu as pltpu

_CB = 8
_M = 8192


def _pool_body(x_ref, g2_ref, g3_ref, t_ref, o_ref):
    x = x_ref[0]
    fe = jnp.dot(x, g2_ref[...], preferred_element_type=jnp.float32)
    fo = jnp.dot(x, g3_ref[...], preferred_element_type=jnp.float32)
    inv = 1.0 / (4.0 * t_ref[:, 0:1])
    ninf = jnp.full((1, 128), -jnp.inf, jnp.float32)

    for c in range(_CB):
        a = fe[c * 64:(c + 1) * 64]
        b = fo[c * 64:(c + 1) * 64]
        ic = inv[c:c + 1, 0:1]
        a_dn = jnp.concatenate([a[1:], ninf], axis=0)
        a_up = jnp.concatenate([ninf, a[:-1]], axis=0)
        b_dn = jnp.concatenate([b[1:], ninf], axis=0)
        b_up = jnp.concatenate([ninf, b[:-1]], axis=0)

        def sh(z, z_dn, z_up, e):
            if e == 0:
                return z
            if e > 0:
                return jnp.concatenate([z[:, e:], z_dn[:, :e]], axis=1)
            return jnp.concatenate([z_up[:, e:], z[:, :e]], axis=1)

        acc = a + (-0.0) * ic
        for e in range(-6, 7):
            if e != 0:
                acc = jnp.maximum(acc, sh(a, a_dn, a_up, e) + (-4.0 * e * e) * ic)
        for e in range(-6, 6):
            w = -float((2 * e + 1) ** 2)
            acc = jnp.maximum(acc, sh(b, b_dn, b_up, e) + w * ic)
        o_ref[0, c] = acc


@jax.jit
def kernel(f, t):
    B, C, N = f.shape
    M = N // 2
    f2 = f.reshape(B, C * 64, 256)
    g2 = (jnp.arange(256)[:, None] == 2 * jnp.arange(128)[None, :]).astype(jnp.float32)
    g3 = (jnp.arange(256)[:, None] == 2 * jnp.arange(128)[None, :] + 1).astype(jnp.float32)
    tb = jnp.broadcast_to(t[:, None], (C, 128))
    grid = (B, C // _CB)
    out = pl.pallas_call(
        _pool_body,
        grid=grid,
        in_specs=[
            pl.BlockSpec((1, _CB * 64, 256), lambda b, c: (b, c, 0)),
            pl.BlockSpec((256, 128), lambda b, c: (0, 0)),
            pl.BlockSpec((256, 128), lambda b, c: (0, 0)),
            pl.BlockSpec((_CB, 128), lambda b, c: (c, 0)),
        ],
        out_specs=pl.BlockSpec((1, _CB, 64, 128), lambda b, c: (b, c, 0, 0)),
        out_shape=jax.ShapeDtypeStruct((B, C, 64, 128), jnp.float32),
        compiler_params=pltpu.CompilerParams(
            dimension_semantics=("parallel", "parallel"),
        ),
    )(f2, g2, g3, tb)
    return out.reshape(B, C, M)


# speedup vs baseline: 2.1478x; 2.1478x over previous
import jax
import jax.numpy as jnp
from jax.experimental import pallas as pl
from jax.experimental.pallas import tpu as pltpu

_M = 8192
_CH = 512


def _pool_body(fe_ref, fo_ref, t_ref, o_ref):
    inv = 1.0 / (4.0 * t_ref[0:1, :])
    for r0 in range(0, _M, _CH):
        acc = fe_ref[0, r0 + 6:r0 + 6 + _CH, :] + (-0.0) * inv
        for e in range(-6, 7):
            if e != 0:
                acc = jnp.maximum(
                    acc, fe_ref[0, r0 + 6 + e:r0 + 6 + e + _CH, :] + (-4.0 * e * e) * inv)
        for e in range(-6, 6):
            w = -float((2 * e + 1) ** 2)
            acc = jnp.maximum(
                acc, fo_ref[0, r0 + 6 + e:r0 + 6 + e + _CH, :] + w * inv)
        o_ref[0, r0:r0 + _CH, :] = acc


@jax.jit
def kernel(f, t):
    B, C, N = f.shape
    M = N // 2
    fr = f.reshape(B, C, M, 2)
    fte = jnp.transpose(fr[..., 0], (0, 2, 1))
    fto = jnp.transpose(fr[..., 1], (0, 2, 1))
    fte_p = jnp.pad(fte, ((0, 0), (6, 10), (0, 0)), constant_values=-jnp.inf)
    fto_p = jnp.pad(fto, ((0, 0), (6, 10), (0, 0)), constant_values=-jnp.inf)
    tb = jnp.broadcast_to(t[None, :], (8, C))
    out_t = pl.pallas_call(
        _pool_body,
        grid=(B,),
        in_specs=[
            pl.BlockSpec((1, M + 16, C), lambda b: (b, 0, 0)),
            pl.BlockSpec((1, M + 16, C), lambda b: (b, 0, 0)),
            pl.BlockSpec((8, C), lambda b: (0, 0)),
        ],
        out_specs=pl.BlockSpec((1, M, C), lambda b: (b, 0, 0)),
        out_shape=jax.ShapeDtypeStruct((B, M, C), jnp.float32),
        compiler_params=pltpu.CompilerParams(
            dimension_semantics=("parallel",),
        ),
    )(fte_p, fto_p, tb)
    return jnp.transpose(out_t, (0, 2, 1))


# speedup vs baseline: 8.8915x; 4.1397x over previous
import jax
import jax.numpy as jnp
from jax.experimental import pallas as pl
from jax.experimental.pallas import tpu as pltpu

_M = 8192
_CH = 512
_SH = _M + 16


def _pool_body(x_ref, t_ref, o_ref, fe_s, fo_s):
    fe_s[0:6] = jnp.full((6, 128), -jnp.inf, jnp.float32)
    fo_s[0:6] = jnp.full((6, 128), -jnp.inf, jnp.float32)
    fe_s[_M + 6:_SH] = jnp.full((10, 128), -jnp.inf, jnp.float32)
    fo_s[_M + 6:_SH] = jnp.full((10, 128), -jnp.inf, jnp.float32)

    io = jax.lax.broadcasted_iota(jnp.int32, (128, 128), 1)
    pidx = jnp.where(io < 64, 2 * io, 2 * (io - 64) + 1)

    for k in range(128):
        v = x_ref[0, :, 128 * k:128 * (k + 1)]
        vp = jnp.take_along_axis(v, pidx, axis=1)
        vt = vp.T
        fe_s[6 + 64 * k:6 + 64 * k + 64] = vt[0:64]
        fo_s[6 + 64 * k:6 + 64 * k + 64] = vt[64:128]

    inv = 1.0 / (4.0 * t_ref[0:1, :])
    for r0 in range(0, _M, _CH):
        acc = fe_s[r0 + 6:r0 + 6 + _CH, :] + (-0.0) * inv
        for e in range(-6, 7):
            if e != 0:
                acc = jnp.maximum(
                    acc, fe_s[r0 + 6 + e:r0 + 6 + e + _CH, :] + (-4.0 * e * e) * inv)
        for e in range(-6, 6):
            w = -float((2 * e + 1) ** 2)
            acc = jnp.maximum(
                acc, fo_s[r0 + 6 + e:r0 + 6 + e + _CH, :] + w * inv)
        for m in range(_CH // 128):
            o_ref[0, :, r0 + 128 * m:r0 + 128 * (m + 1)] = acc[128 * m:128 * (m + 1)].T


@jax.jit
def kernel(f, t):
    B, C, N = f.shape
    M = N // 2
    tb = jnp.broadcast_to(t[None, :], (8, C))
    return pl.pallas_call(
        _pool_body,
        grid=(B,),
        in_specs=[
            pl.BlockSpec((1, C, N), lambda b: (b, 0, 0)),
            pl.BlockSpec((8, C), lambda b: (0, 0)),
        ],
        out_specs=pl.BlockSpec((1, C, M), lambda b: (b, 0, 0)),
        out_shape=jax.ShapeDtypeStruct((B, C, M), jnp.float32),
        scratch_shapes=[
            pltpu.VMEM((_SH, 128), jnp.float32),
            pltpu.VMEM((_SH, 128), jnp.float32),
        ],
        compiler_params=pltpu.CompilerParams(
            dimension_semantics=("parallel",),
            vmem_limit_bytes=50 * 1024 * 1024,
        ),
    )(f, tb)


# speedup vs baseline: 9.0837x; 1.0216x over previous
import jax
import jax.numpy as jnp
from jax.experimental import pallas as pl
from jax.experimental.pallas import tpu as pltpu

_M = 8192
_CH = 1024
_SH = _M + 16


def _pool_body(x_ref, t_ref, o_ref, fe_s, fo_s):
    fe_s[0:6] = jnp.full((6, 128), -jnp.inf, jnp.float32)
    fo_s[0:6] = jnp.full((6, 128), -jnp.inf, jnp.float32)
    fe_s[_M + 6:_SH] = jnp.full((10, 128), -jnp.inf, jnp.float32)
    fo_s[_M + 6:_SH] = jnp.full((10, 128), -jnp.inf, jnp.float32)

    io = jax.lax.broadcasted_iota(jnp.int32, (128, 128), 1)
    pidx = jnp.where(io < 64, 2 * io, 2 * (io - 64) + 1)

    for k in range(128):
        v = x_ref[0, :, 128 * k:128 * (k + 1)]
        vp = jnp.take_along_axis(v, pidx, axis=1)
        vt = vp.T
        fe_s[6 + 64 * k:6 + 64 * k + 64] = vt[0:64]
        fo_s[6 + 64 * k:6 + 64 * k + 64] = vt[64:128]

    inv = 1.0 / (4.0 * t_ref[0:1, :])
    for r0 in range(0, _M, _CH):
        acc = fe_s[r0 + 6:r0 + 6 + _CH, :]
        for e in range(-6, 7):
            if e != 0:
                acc = jnp.maximum(
                    acc, fe_s[r0 + 6 + e:r0 + 6 + e + _CH, :] + (-4.0 * e * e) * inv)
        for e in range(-6, 6):
            w = -float((2 * e + 1) ** 2)
            acc = jnp.maximum(
                acc, fo_s[r0 + 6 + e:r0 + 6 + e + _CH, :] + w * inv)
        for m in range(_CH // 128):
            o_ref[0, :, r0 + 128 * m:r0 + 128 * (m + 1)] = acc[128 * m:128 * (m + 1)].T


@jax.jit
def kernel(f, t):
    B, C, N = f.shape
    M = N // 2
    tb = jnp.broadcast_to(t[None, :], (8, C))
    return pl.pallas_call(
        _pool_body,
        grid=(B,),
        in_specs=[
            pl.BlockSpec((1, C, N), lambda b: (b, 0, 0)),
            pl.BlockSpec((8, C), lambda b: (0, 0)),
        ],
        out_specs=pl.BlockSpec((1, C, M), lambda b: (b, 0, 0)),
        out_shape=jax.ShapeDtypeStruct((B, C, M), jnp.float32),
        scratch_shapes=[
            pltpu.VMEM((_SH, 128), jnp.float32),
            pltpu.VMEM((_SH, 128), jnp.float32),
        ],
        compiler_params=pltpu.CompilerParams(
            dimension_semantics=("parallel",),
            vmem_limit_bytes=50 * 1024 * 1024,
        ),
    )(f, tb)


# speedup vs baseline: 10.5348x; 1.1598x over previous
import jax
import jax.numpy as jnp
from jax.experimental import pallas as pl
from jax.experimental.pallas import tpu as pltpu

_M = 8192
_CH = 1024
_SH = _M + 16


def _pool_body(x_ref, t_ref, o_ref, fe_s, fo_s):
    fe_s[0:6] = jnp.full((6, 128), -jnp.inf, jnp.float32)
    fo_s[0:6] = jnp.full((6, 128), -jnp.inf, jnp.float32)
    fe_s[_M + 6:_SH] = jnp.full((10, 128), -jnp.inf, jnp.float32)
    fo_s[_M + 6:_SH] = jnp.full((10, 128), -jnp.inf, jnp.float32)

    io = jax.lax.broadcasted_iota(jnp.int32, (128, 128), 1)
    pidx = jnp.where(io < 64, 2 * io, 2 * (io - 64) + 1)

    for k in range(128):
        v = x_ref[0, :, 128 * k:128 * (k + 1)]
        vp = jnp.take_along_axis(v, pidx, axis=1)
        vt = vp.T
        fe_s[6 + 64 * k:6 + 64 * k + 64] = vt[0:64]
        fo_s[6 + 64 * k:6 + 64 * k + 64] = vt[64:128]

    inv = 1.0 / (4.0 * t_ref[0:1, :])
    for r0 in range(0, _M, _CH):
        base = r0 + 6

        def ld(ref, e):
            return ref[base + e:base + e + _CH, :]

        acc = ld(fe_s, 0)
        for e in range(1, 7):
            pair = jnp.maximum(ld(fe_s, e), ld(fe_s, -e))
            acc = jnp.maximum(acc, pair + (-4.0 * e * e) * inv)
        for o in range(1, 13, 2):
            pair = jnp.maximum(ld(fo_s, (o - 1) // 2), ld(fo_s, -(o + 1) // 2))
            acc = jnp.maximum(acc, pair + float(-o * o) * inv)
        for m in range(_CH // 128):
            o_ref[0, :, r0 + 128 * m:r0 + 128 * (m + 1)] = acc[128 * m:128 * (m + 1)].T


@jax.jit
def kernel(f, t):
    B, C, N = f.shape
    M = N // 2
    tb = jnp.broadcast_to(t[None, :], (8, C))
    return pl.pallas_call(
        _pool_body,
        grid=(B,),
        in_specs=[
            pl.BlockSpec((1, C, N), lambda b: (b, 0, 0)),
            pl.BlockSpec((8, C), lambda b: (0, 0)),
        ],
        out_specs=pl.BlockSpec((1, C, M), lambda b: (b, 0, 0)),
        out_shape=jax.ShapeDtypeStruct((B, C, M), jnp.float32),
        scratch_shapes=[
            pltpu.VMEM((_SH, 128), jnp.float32),
            pltpu.VMEM((_SH, 128), jnp.float32),
        ],
        compiler_params=pltpu.CompilerParams(
            dimension_semantics=("parallel",),
            vmem_limit_bytes=50 * 1024 * 1024,
        ),
    )(f, tb)


# speedup vs baseline: 10.9281x; 1.0373x over previous
import jax
import jax.numpy as jnp
from jax.experimental import pallas as pl
from jax.experimental.pallas import tpu as pltpu

_M = 8192
_CH = 1024
_SH = _M + 16


def _pool_body(x_ref, t_ref, o_ref, fe_s, fo_s):
    fe_s[0:6] = jnp.full((6, 128), -jnp.inf, jnp.float32)
    fo_s[0:6] = jnp.full((6, 128), -jnp.inf, jnp.float32)
    fe_s[_M + 6:_SH] = jnp.full((10, 128), -jnp.inf, jnp.float32)
    fo_s[_M + 6:_SH] = jnp.full((10, 128), -jnp.inf, jnp.float32)

    io = jax.lax.broadcasted_iota(jnp.int32, (128, 128), 1)
    pidx = jnp.where(io < 64, 2 * io, 2 * (io - 64) + 1)

    for k in range(128):
        v = x_ref[0, :, 128 * k:128 * (k + 1)]
        vp = jnp.take_along_axis(v, pidx, axis=1)
        vt = vp.T
        fe_s[6 + 64 * k:6 + 64 * k + 64] = vt[0:64]
        fo_s[6 + 64 * k:6 + 64 * k + 64] = vt[64:128]

    inv = 1.0 / (4.0 * t_ref[0:1, :])
    for r0 in range(0, _M, _CH):
        base = r0 + 6

        def ld(ref, e):
            return ref[base + e:base + e + _CH, :]

        acc = ld(fe_s, 0)
        for e in (4, 2, 6, 3, 5, 1):
            pair = jnp.maximum(ld(fe_s, e), ld(fe_s, -e))
            acc = jnp.maximum(acc, pair + (-4.0 * e * e) * inv)
        for o in (5, 11, 7, 9, 1, 3):
            pair = jnp.maximum(ld(fo_s, (o - 1) // 2), ld(fo_s, -(o + 1) // 2))
            acc = jnp.maximum(acc, pair + float(-o * o) * inv)
        for m in range(_CH // 128):
            o_ref[0, :, r0 + 128 * m:r0 + 128 * (m + 1)] = acc[128 * m:128 * (m + 1)].T


@jax.jit
def kernel(f, t):
    B, C, N = f.shape
    M = N // 2
    tb = jnp.broadcast_to(t[None, :], (8, C))
    return pl.pallas_call(
        _pool_body,
        grid=(B,),
        in_specs=[
            pl.BlockSpec((1, C, N), lambda b: (b, 0, 0)),
            pl.BlockSpec((8, C), lambda b: (0, 0)),
        ],
        out_specs=pl.BlockSpec((1, C, M), lambda b: (b, 0, 0)),
        out_shape=jax.ShapeDtypeStruct((B, C, M), jnp.float32),
        scratch_shapes=[
            pltpu.VMEM((_SH, 128), jnp.float32),
            pltpu.VMEM((_SH, 128), jnp.float32),
        ],
        compiler_params=pltpu.CompilerParams(
            dimension_semantics=("parallel",),
            vmem_limit_bytes=50 * 1024 * 1024,
        ),
    )(f, tb)
